# Initial kernel scaffold; baseline (speedup 1.0000x reference)
import jax
import jax.numpy as jnp
from jax.experimental import pallas as pl


def kernel(t, t_knots, values):
    raise NotImplementedError("write your pallas kernel here")


# speedup vs baseline: 4.3486x; 4.3486x over previous
import functools

import jax
import jax.numpy as jnp
from jax import lax
from jax.experimental import pallas as pl
from jax.experimental.pallas import tpu as pltpu
from jax.experimental.pallas import tpu_sc as plsc

_LANES = 16


@functools.lru_cache(maxsize=None)
def _build(n, k, k_pad):
    info = plsc.get_sparse_core_info()
    nc, ns = info.num_cores, info.num_subcores
    nw = nc * ns
    chunk = n // nw
    nvec = chunk // _LANES

    @functools.partial(
        pl.kernel,
        out_type=jax.ShapeDtypeStruct((n,), jnp.float32),
        mesh=plsc.VectorSubcoreMesh(core_axis_name="c", subcore_axis_name="s"),
        compiler_params=pltpu.CompilerParams(needs_layout_passes=False),
        scratch_types=[
            pltpu.VMEM((chunk,), jnp.float32),
            pltpu.VMEM((k_pad,), jnp.float32),
            pltpu.VMEM((k_pad,), jnp.float32),
            pltpu.VMEM((chunk,), jnp.float32),
        ],
    )
    def run(t_hbm, kn_hbm, va_hbm, out_hbm, t_v, kn_v, va_v, o_v):
        wid = lax.axis_index("s") * nc + lax.axis_index("c")
        base = wid * chunk
        pltpu.sync_copy(kn_hbm, kn_v)
        pltpu.sync_copy(va_hbm, va_v)
        pltpu.sync_copy(t_hbm.at[pl.ds(base, chunk)], t_v)
        kb = [plsc.load_gather(kn_v, [jnp.full((_LANES,), j, jnp.int32)])
              for j in range(1, k)]
        one = jnp.ones((_LANES,), jnp.int32)
        for i in range(nvec):
            tv = t_v[pl.ds(i * _LANES, _LANES)]
            cnt = one
            for j in range(1, k):
                cnt = cnt + jnp.where(kb[j - 1] < tv, one, 0)
            idx1 = jnp.minimum(jnp.maximum(cnt, 1), k - 1)
            idx0 = idx1 - 1
            t0 = plsc.load_gather(kn_v, [idx0])
            t1 = plsc.load_gather(kn_v, [idx1])
            v0 = plsc.load_gather(va_v, [idx0])
            v1 = plsc.load_gather(va_v, [idx1])
            w = (tv - t0) / (t1 - t0)
            o_v[pl.ds(i * _LANES, _LANES)] = (1.0 - w) * v0 + w * v1
        pltpu.sync_copy(o_v, out_hbm.at[pl.ds(base, chunk)])

    return run


def kernel(t, t_knots, values):
    t = jnp.asarray(t, jnp.float32).reshape(-1)
    n = t.shape[0]
    k = t_knots.shape[0]
    k_pad = -(-k // _LANES) * _LANES
    pad = k_pad - k
    kn = jnp.concatenate([t_knots.astype(jnp.float32),
                          jnp.zeros((pad,), jnp.float32)])
    va = jnp.concatenate([values.astype(jnp.float32),
                          jnp.zeros((pad,), jnp.float32)])
    return _build(n, k, k_pad)(t, kn, va)
